# baseline (speedup 1.0000x reference)
import functools

import jax
import jax.numpy as jnp
from jax import lax
from jax.experimental import pallas as pl
from jax.experimental.pallas import tpu as pltpu
from jax.experimental.pallas import tpu_sc as plsc

N = 10000
H = 128
OUT = 40
NC = 2
NS = 16
NW = NC * NS
CH = 128
N_PAD = 10240
ROWS = N_PAD // NS
DEG_W = 16
ZB = 64


NBUF = 2
SB = 8
DEGB = 4


def _fill(ref, nrows, val):
    @pl.loop(0, nrows)
    def _(i):
        @pl.loop(0, H, step=16)
        def _(j):
            ref[i, pl.ds(j, 16)] = jnp.full((16,), val, jnp.float32)


def _sc_agg_build(cpw, gather):
    mesh = plsc.VectorSubcoreMesh(core_axis_name="c", subcore_axis_name="s")
    if gather:
        assert cpw % SB == 0
        nsb = cpw // SB
        scratch = [
            pltpu.VMEM((SB, CH), jnp.int32),
            pltpu.VMEM((SB, CH), jnp.int32),
            pltpu.VMEM((CH, H), jnp.float32),
            pltpu.VMEM((CH, H), jnp.float32),
            pltpu.VMEM_SHARED((N_PAD, H), jnp.float32),
            pltpu.SemaphoreType.DMA, pltpu.SemaphoreType.DMA,
            pltpu.SemaphoreType.DMA, pltpu.SemaphoreType.DMA,
        ]
    else:
        assert cpw % DEGB == 0
        scratch = [
            pltpu.VMEM((cpw, CH), jnp.int32),
            pltpu.VMEM((CH, H), jnp.float32),
            pltpu.VMEM_SHARED((N_PAD, H), jnp.float32),
        ] + [pltpu.SemaphoreType.DMA for _ in range(DEGB)]

    def body(refs):
        if gather:
            (h_hbm, src_hbm, dst_hbm, out_hbm,
             didx_b, sidx_b, r0, r1, acc, g0, g1, s0, s1) = refs
            rows, gsems, ssems = [r0, r1], [g0, g1], [s0, s1]
        else:
            dst_hbm, out_hbm, didx_all, ones_v, acc = refs[:5]
            ssems = list(refs[5:])
        c = lax.axis_index("c")
        s = lax.axis_index("s")
        wid = s * NC + c
        base = s * ROWS

        zbuf = rows[0] if gather else ones_v
        _fill(zbuf, CH, 0.0)

        @pl.loop(0, ROWS, step=CH)
        def _(r):
            pltpu.sync_copy(zbuf, acc.at[pl.ds(base + r, CH)])

        if not gather:
            _fill(ones_v, CH, 1.0)
            pltpu.sync_copy(dst_hbm.at[pl.ds(wid * cpw, cpw)], didx_all)

        plsc.subcore_barrier()

        if gather:
            @pl.loop(0, nsb)
            def _(q):
                @pl.when(q > 0)
                def _():
                    for b in range(2):
                        pltpu.make_async_copy(
                            rows[b], acc.at[didx_b.at[SB - 2 + b]],
                            ssems[b]).wait()
                qb = (wid * nsb + q) * SB
                pltpu.sync_copy(dst_hbm.at[pl.ds(qb, SB)], didx_b)
                pltpu.sync_copy(src_hbm.at[pl.ds(qb, SB)], sidx_b)
                gh = [None, None]
                gh[0] = pltpu.async_copy(h_hbm.at[sidx_b.at[0]], rows[0],
                                         gsems[0])
                for k in range(SB):
                    b = k % 2
                    nb = (k + 1) % 2
                    if k + 1 < SB:
                        if k >= 1:
                            pltpu.make_async_copy(
                                rows[nb], acc.at[didx_b.at[k - 1]],
                                ssems[nb]).wait()
                        gh[nb] = pltpu.async_copy(
                            h_hbm.at[sidx_b.at[k + 1]], rows[nb], gsems[nb])
                    gh[b].wait()
                    pltpu.async_copy(rows[b], acc.at[didx_b.at[k]],
                                     ssems[b], add=True)

            for b in range(2):
                pltpu.make_async_copy(rows[b], acc.at[didx_b.at[SB - 2 + b]],
                                      ssems[b]).wait()
        else:
            @pl.loop(0, cpw, step=DEGB)
            def _(j):
                for b in range(DEGB):
                    @pl.when(j > 0)
                    def _(b=b):
                        pltpu.make_async_copy(
                            ones_v, acc.at[didx_all.at[j - DEGB + b]],
                            ssems[b]).wait()
                    pltpu.async_copy(ones_v, acc.at[didx_all.at[j + b]],
                                     ssems[b], add=True)

            for b in range(DEGB):
                pltpu.make_async_copy(
                    ones_v, acc.at[didx_all.at[cpw - DEGB + b]],
                    ssems[b]).wait()

        plsc.subcore_barrier()
        pltpu.sync_copy(acc.at[pl.ds(base, ROWS)],
                        out_hbm.at[pl.ds(c * N_PAD + base, ROWS)])

    out_type = jax.ShapeDtypeStruct((NC * N_PAD, H), jnp.float32)

    @functools.partial(pl.kernel, out_type=out_type, mesh=mesh,
                       scratch_types=scratch)
    def k(*refs):
        body(refs)

    return k


def _sc_degree(dst2d, cpw):
    return _sc_agg_build(cpw, gather=False)(dst2d)


def _sc_agg(h, src2d, dst2d, cpw):
    return _sc_agg_build(cpw, gather=True)(h, src2d, dst2d)


BR = 400


def _tc_layer_body(a0, a1, d0, d1, h_ref, wl, blr, wr, o_ref):
    cnt = d0[:, 0:1] + d1[:, 0:1]
    inv = 1.0 / jnp.maximum(cnt, 1.0)
    mean = (a0[...] + a1[...]) * inv
    acc = lax.dot_general(mean, wl[...], (((1,), (1,)), ((), ())),
                          preferred_element_type=jnp.float32,
                          precision=lax.Precision.HIGHEST)
    acc = acc + blr[...]
    acc = acc + lax.dot_general(h_ref[...], wr[...], (((1,), (1,)), ((), ())),
                                preferred_element_type=jnp.float32,
                                precision=lax.Precision.HIGHEST)
    o_ref[...] = jnp.maximum(acc, 0.0)


def _tc_layer(a0, a1, d0, d1, h, Wl, bl, Wr):
    nb = N // BR
    return pl.pallas_call(
        _tc_layer_body,
        grid=(nb,),
        in_specs=[
            pl.BlockSpec((BR, H), lambda i: (i, 0)),
            pl.BlockSpec((BR, H), lambda i: (i, 0)),
            pl.BlockSpec((BR, H), lambda i: (i, 0)),
            pl.BlockSpec((BR, H), lambda i: (i, 0)),
            pl.BlockSpec((BR, H), lambda i: (i, 0)),
            pl.BlockSpec((H, H), lambda i: (0, 0)),
            pl.BlockSpec((1, H), lambda i: (0, 0)),
            pl.BlockSpec((H, H), lambda i: (0, 0)),
        ],
        out_specs=pl.BlockSpec((BR, H), lambda i: (i, 0)),
        out_shape=jax.ShapeDtypeStruct((N, H), jnp.float32),
    )(a0, a1, d0, d1, h, Wl, bl, Wr)


def _tc_final_body(h1, h2, h3, w1, w2, w3, br, o_ref):
    acc = lax.dot_general(h1[...], w1[...], (((1,), (1,)), ((), ())),
                          preferred_element_type=jnp.float32,
                          precision=lax.Precision.HIGHEST)
    acc = acc + lax.dot_general(h2[...], w2[...], (((1,), (1,)), ((), ())),
                                preferred_element_type=jnp.float32,
                                precision=lax.Precision.HIGHEST)
    acc = acc + lax.dot_general(h3[...], w3[...], (((1,), (1,)), ((), ())),
                                preferred_element_type=jnp.float32,
                                precision=lax.Precision.HIGHEST)
    o_ref[...] = acc + br[...]


def _tc_final(h1, h2, h3, w1, w2, w3, fc_b):
    nb = N // BR
    return pl.pallas_call(
        _tc_final_body,
        grid=(nb,),
        in_specs=[
            pl.BlockSpec((BR, H), lambda i: (i, 0)),
            pl.BlockSpec((BR, H), lambda i: (i, 0)),
            pl.BlockSpec((BR, H), lambda i: (i, 0)),
            pl.BlockSpec((OUT, H), lambda i: (0, 0)),
            pl.BlockSpec((OUT, H), lambda i: (0, 0)),
            pl.BlockSpec((OUT, H), lambda i: (0, 0)),
            pl.BlockSpec((1, OUT), lambda i: (0, 0)),
        ],
        out_specs=pl.BlockSpec((BR, OUT), lambda i: (i, 0)),
        out_shape=jax.ShapeDtypeStruct((N, OUT), jnp.float32),
    )(h1, h2, h3, w1, w2, w3, fc_b)


def kernel(x, edge_index, Wl0, bl0, Wr0, Wl1, bl1, Wr1, Wl2, bl2, Wr2, fc_W, fc_b):
    src = edge_index[0]
    dst = edge_index[1]
    e = src.shape[0]
    cpw = -(-e // (NW * CH))
    cpw = -(-cpw // SB) * SB
    e_pad = NW * CH * cpw
    src_p = jnp.concatenate(
        [src, jnp.zeros((e_pad - e,), jnp.int32)]).reshape(-1, CH)
    dst_p = jnp.concatenate(
        [dst, jnp.full((e_pad - e,), N, jnp.int32)]).reshape(-1, CH)

    degs = _sc_degree(dst_p, cpw)
    d0 = degs[0:N]
    d1 = degs[N_PAD:N_PAD + N]

    h = x
    hs = []
    for (Wl, bl, Wr) in ((Wl0, bl0, Wr0), (Wl1, bl1, Wr1), (Wl2, bl2, Wr2)):
        parts = _sc_agg(h, src_p, dst_p, cpw)
        h = _tc_layer(parts[0:N], parts[N_PAD:N_PAD + N], d0, d1, h,
                      Wl, bl.reshape(1, H), Wr)
        hs.append(h)

    return _tc_final(hs[0], hs[1], hs[2],
                     fc_W[:, 0:H], fc_W[:, H:2 * H], fc_W[:, 2 * H:3 * H],
                     fc_b.reshape(1, OUT))
